# Initial kernel scaffold; baseline (speedup 1.0000x reference)
import jax
import jax.numpy as jnp
from jax.experimental import pallas as pl


def kernel(le_0, le_1, idx0_2__0_0, idx1_2__0_0, mult_2__0_0, gcg_2__0_0, idx0_2__0_1, idx1_2__0_1, mult_2__0_1, gcg_2__0_1, idx0_2__1_0, idx1_2__1_0, mult_2__1_0, idx0_2__1_1, idx1_2__1_1, mult_2__1_1, gcg_2__1_1, idx0_3__0_0_0, idx1_3__0_0_0, mult_3__0_0_0, gcg_3__0_0_0, idx0_3__0_0_1, idx1_3__0_0_1, mult_3__0_0_1, gcg_3__0_0_1, idx0_3__0_1_1, idx1_3__0_1_1, mult_3__0_1_1, gcg_3__0_1_1, idx0_3__1_0_0, idx1_3__1_0_0, mult_3__1_0_0, gcg_3__1_0_0, idx0_3__1_0_1, idx1_3__1_0_1, mult_3__1_0_1, idx0_3__1_1_0, idx1_3__1_1_0, mult_3__1_1_0, gcg_3__1_1_0, idx0_3__1_1_1, idx1_3__1_1_1, mult_3__1_1_1, gcg_3__1_1_1):
    raise NotImplementedError("write your pallas kernel here")


# speedup vs baseline: 5.1059x; 5.1059x over previous
import jax
import jax.numpy as jnp
from jax.experimental import pallas as pl
from jax.experimental.pallas import tpu as pltpu

_N = 393216
_LANES = 128
_S = 16
_NB = _N // (_S * _LANES)

_K2 = 12
_K3 = 28

_NU2 = [
    ((0, 0), 1, 1, True),
    ((0, 1), 1, 3, True),
    ((1, 0), 3, 1, False),
    ((1, 1), 3, 3, True),
]
_M2 = {t: mp * ml for (t, mp, ml, _) in _NU2}

_NU3 = [
    ((0, 0, 0), (0, 0), 1, 1),
    ((0, 0, 1), (0, 0), 3, 2),
    ((0, 1, 1), (0, 1), 3, 1),
    ((1, 0, 0), (1, 0), 1, 2),
    ((1, 1, 0), (1, 1), 1, 1),
    ((1, 1, 1), (1, 1), 3, 2),
]


def _le_rows(le0, le1, l, i, ml):
    if l == 0:
        return [le0[pl.ds(i, 1), 0][0]]
    v = le1[pl.ds(i * 3, 3), 0]
    return [v[q] for q in range(ml)]


def _body(*refs):
    it = iter(refs)
    le0 = next(it)
    le1 = next(it)
    nu2_refs = []
    for (_, _, _, has_cg) in _NU2:
        i0r, i1r = next(it), next(it)
        mur, gr = (next(it), next(it)) if has_cg else (None, None)
        nu2_refs.append((i0r, i1r, mur, gr))
    nu3_refs = [(next(it), next(it), next(it), next(it)) for _ in _NU3]
    outs2 = [next(it) for (_, _, _, has_cg) in _NU2 if has_cg]
    outs3 = [next(it) for _ in _NU3]
    scr = {t: next(it) for (t, _, _, _) in _NU2}

    o2 = iter(outs2)
    for (t, mp, ml, has_cg), (i0r, i1r, mur, gr) in zip(_NU2, nu2_refs):
        m2 = mp * ml
        scr_t = scr[t]
        out = next(o2) if has_cg else None
        for k in range(_K2):
            i0, i1 = i0r[k], i1r[k]
            ar = _le_rows(le0, le1, t[0], i0, mp)
            br = _le_rows(le0, le1, t[1], i1, ml)
            rows = [ap * bq for ap in ar for bq in br]
            for m in range(m2):
                scr_t[k, m] = rows[m]
            if has_cg:
                mk = mur[k]
                for r in range(2):
                    acc = rows[0] * gr[r, 0]
                    for m in range(1, m2):
                        acc = acc + rows[m] * gr[r, m]
                    out[r * _K2 + k, 0] = acc * mk

    for (t, pfx, ml, nr), (i0r, i1r, mur, gr), out in zip(_NU3, nu3_refs,
                                                          outs3):
        mp = _M2[pfx]
        scr_p = scr[pfx]
        for k in range(_K3):
            i0, i1 = i0r[k], i1r[k]
            a = scr_p[pl.ds(i0, 1)][0]
            ar = [a[p] for p in range(mp)]
            br = _le_rows(le0, le1, t[2], i1, ml)
            mk = mur[k]
            for r in range(nr):
                if ml == 1:
                    acc = ar[0] * gr[r, 0]
                    for p in range(1, mp):
                        acc = acc + ar[p] * gr[r, p]
                    acc = acc * br[0]
                else:
                    acc = None
                    for p in range(mp):
                        inner = br[0] * gr[r, p * ml]
                        for q in range(1, ml):
                            inner = inner + br[q] * gr[r, p * ml + q]
                        term = ar[p] * inner
                        acc = term if acc is None else acc + term
                out[r * _K3 + k, 0] = acc * mk


def kernel(
    le_0, le_1,
    idx0_2__0_0, idx1_2__0_0, mult_2__0_0, gcg_2__0_0,
    idx0_2__0_1, idx1_2__0_1, mult_2__0_1, gcg_2__0_1,
    idx0_2__1_0, idx1_2__1_0, mult_2__1_0,
    idx0_2__1_1, idx1_2__1_1, mult_2__1_1, gcg_2__1_1,
    idx0_3__0_0_0, idx1_3__0_0_0, mult_3__0_0_0, gcg_3__0_0_0,
    idx0_3__0_0_1, idx1_3__0_0_1, mult_3__0_0_1, gcg_3__0_0_1,
    idx0_3__0_1_1, idx1_3__0_1_1, mult_3__0_1_1, gcg_3__0_1_1,
    idx0_3__1_0_0, idx1_3__1_0_0, mult_3__1_0_0, gcg_3__1_0_0,
    idx0_3__1_0_1, idx1_3__1_0_1, mult_3__1_0_1,
    idx0_3__1_1_0, idx1_3__1_1_0, mult_3__1_1_0, gcg_3__1_1_0,
    idx0_3__1_1_1, idx1_3__1_1_1, mult_3__1_1_1, gcg_3__1_1_1,
):
    nu2_ops = [
        (idx0_2__0_0, idx1_2__0_0, mult_2__0_0, gcg_2__0_0),
        (idx0_2__0_1, idx1_2__0_1, mult_2__0_1, gcg_2__0_1),
        (idx0_2__1_0, idx1_2__1_0, None, None),
        (idx0_2__1_1, idx1_2__1_1, mult_2__1_1, gcg_2__1_1),
    ]
    nu3_ops = [
        (idx0_3__0_0_0, idx1_3__0_0_0, mult_3__0_0_0, gcg_3__0_0_0),
        (idx0_3__0_0_1, idx1_3__0_0_1, mult_3__0_0_1, gcg_3__0_0_1),
        (idx0_3__0_1_1, idx1_3__0_1_1, mult_3__0_1_1, gcg_3__0_1_1),
        (idx0_3__1_0_0, idx1_3__1_0_0, mult_3__1_0_0, gcg_3__1_0_0),
        (idx0_3__1_1_0, idx1_3__1_1_0, mult_3__1_1_0, gcg_3__1_1_0),
        (idx0_3__1_1_1, idx1_3__1_1_1, mult_3__1_1_1, gcg_3__1_1_1),
    ]

    smem = pl.BlockSpec(memory_space=pltpu.MemorySpace.SMEM)

    def lane_spec(rows):
        return pl.BlockSpec((rows, 1, _S, _LANES), lambda j: (0, j, 0, 0))

    flat_in = [le_0.reshape(4, _NB, _S, _LANES),
               le_1.reshape(12, _NB, _S, _LANES)]
    in_specs = [lane_spec(4), lane_spec(12)]
    for (i0, i1, mu, g), (_, _, _, has_cg) in zip(nu2_ops, _NU2):
        flat_in += [i0, i1]
        in_specs += [smem, smem]
        if has_cg:
            flat_in += [mu, g]
            in_specs += [smem, smem]
    for i0, i1, mu, g in nu3_ops:
        flat_in += [i0, i1, mu, g]
        in_specs += [smem, smem, smem, smem]

    out_shape, out_specs = [], []
    for (t, mp, ml, has_cg) in _NU2:
        if has_cg:
            out_shape.append(
                jax.ShapeDtypeStruct((2 * _K2, _NB, _S, _LANES), jnp.float32))
            out_specs.append(lane_spec(2 * _K2))
    for (t, pfx, ml, nr) in _NU3:
        out_shape.append(
            jax.ShapeDtypeStruct((nr * _K3, _NB, _S, _LANES), jnp.float32))
        out_specs.append(lane_spec(nr * _K3))

    scratch_shapes = [pltpu.VMEM((_K2, _M2[t], _S, _LANES), jnp.float32)
                      for (t, _, _, _) in _NU2]

    outs = pl.pallas_call(
        _body,
        grid=(_NB,),
        in_specs=in_specs,
        out_specs=out_specs,
        out_shape=out_shape,
        scratch_shapes=scratch_shapes,
        compiler_params=pltpu.CompilerParams(
            dimension_semantics=("parallel",),
            vmem_limit_bytes=56 * 2**20),
        cost_estimate=pl.CostEstimate(
            flops=2 * 2000 * _N, transcendentals=0,
            bytes_accessed=4 * _N * (16 + 324)),
    )(*flat_in)

    i = 0
    d2 = {}
    for (t, mp, ml, has_cg) in _NU2:
        if has_cg:
            d2[t] = outs[i].reshape(2, _K2, _N)
            i += 1
    d3 = {}
    for (t, pfx, ml, nr) in _NU3:
        d3[t] = outs[i].reshape(nr, _K3, _N)
        i += 1
    return {(0, 1): [dict(), dict(), d2, d3]}
